# Initial kernel scaffold; baseline (speedup 1.0000x reference)
import jax
import jax.numpy as jnp
from jax.experimental import pallas as pl


def kernel(inputs, decoder_segment_ids, decoder_positions, gate_kernel, wi_0, wi_1, wo):
    raise NotImplementedError("write your pallas kernel here")


# speedup vs baseline: 1.0818x; 1.0818x over previous
import functools

import jax
import jax.numpy as jnp
from jax.experimental import pallas as pl
from jax.experimental.pallas import tpu as pltpu

TILE = 256
FB = 512


def _router_kernel(x_ref, g_ref, idx_ref, w_ref, *, n_exp):
    logits = jnp.dot(x_ref[...], g_ref[...], preferred_element_type=jnp.float32)
    eidx = jax.lax.broadcasted_iota(jnp.int32, logits.shape, 1)
    m1 = jnp.max(logits, axis=1, keepdims=True)
    i1 = jnp.min(jnp.where(logits == m1, eidx, n_exp), axis=1, keepdims=True)
    masked = jnp.where(eidx == i1, -jnp.inf, logits)
    m2 = jnp.max(masked, axis=1, keepdims=True)
    i2 = jnp.min(jnp.where(masked == m2, eidx, n_exp), axis=1, keepdims=True)
    e2 = jnp.exp(m2 - m1)
    w1 = 1.0 / (1.0 + e2)
    w2 = e2 / (1.0 + e2)
    idx_ref[...] = jnp.concatenate([i1, i2], axis=1)
    w_ref[...] = jnp.concatenate([w1, w2], axis=1)


def _gmm_kernel(grp_ref, tok_ref,
                x_ref, wi0_ref, wi1_ref, wo_ref, w_ref,
                out_ref,
                xs_scr, acc_scr, row_scr, *, nf):
    i = pl.program_id(0)
    f = pl.program_id(1)

    @pl.when(jnp.logical_and(i == 0, f == 0))
    def _():
        out_ref[...] = jnp.zeros_like(out_ref)

    @pl.when(f == 0)
    def _():
        def gather_body(r, _):
            tok = tok_ref[i * TILE + r]
            xs_scr[r, :] = x_ref[tok, :]
            return 0
        jax.lax.fori_loop(0, TILE, gather_body, 0, unroll=8)

    xs = xs_scr[...]
    a0 = jnp.dot(xs, wi0_ref[0], preferred_element_type=jnp.float32)
    a1 = jnp.dot(xs, wi1_ref[0], preferred_element_type=jnp.float32)
    h = jax.nn.gelu(a0) * a1
    contrib = jnp.dot(h, wo_ref[0], preferred_element_type=jnp.float32)

    @pl.when(f == 0)
    def _():
        acc_scr[...] = contrib

    @pl.when(f != 0)
    def _():
        acc_scr[...] += contrib

    @pl.when(f == nf - 1)
    def _():
        row_scr[...] = acc_scr[...] * w_ref[...]

        def scatter_body(r, _):
            tok = tok_ref[i * TILE + r]
            out_ref[tok, :] += row_scr[r, :]
            return 0
        jax.lax.fori_loop(0, TILE, scatter_body, 0, unroll=8)


def kernel(inputs, decoder_segment_ids, decoder_positions, gate_kernel, wi_0, wi_1, wo):
    del decoder_segment_ids, decoder_positions
    b, s, d = inputs.shape
    t = b * s
    n_exp = gate_kernel.shape[-1]
    f_dim = wi_0.shape[-1]
    k = 2
    nf = f_dim // FB
    nt = (t * k) // TILE + n_exp
    padrows = nt * TILE

    x = inputs.reshape(t, d)

    top_idx, top_w = pl.pallas_call(
        functools.partial(_router_kernel, n_exp=n_exp),
        out_shape=(
            jax.ShapeDtypeStruct((t, k), jnp.int32),
            jax.ShapeDtypeStruct((t, k), jnp.float32),
        ),
    )(x, gate_kernel)

    flat_e = top_idx.reshape(-1)
    flat_t = (jnp.arange(t * k, dtype=jnp.int32) // k)
    flat_w = top_w.reshape(-1)
    counts = jnp.bincount(flat_e, length=n_exp)
    padded = ((counts + TILE - 1) // TILE) * TILE
    pend = jnp.cumsum(padded)
    pstart = pend - padded
    ustart = jnp.cumsum(counts) - counts
    order = jnp.argsort(flat_e, stable=True)
    se = flat_e[order]
    pos = jnp.arange(t * k)
    dest = pstart[se] + (pos - ustart[se])
    sorted_tok = jnp.zeros(padrows, jnp.int32).at[dest].set(flat_t[order])
    sorted_w = jnp.zeros(padrows, jnp.float32).at[dest].set(flat_w[order])
    tile_grp = jnp.clip(
        jnp.searchsorted(pend, jnp.arange(nt) * TILE, side='right'),
        0, n_exp - 1).astype(jnp.int32)

    grid_spec = pltpu.PrefetchScalarGridSpec(
        num_scalar_prefetch=2,
        grid=(nt, nf),
        in_specs=[
            pl.BlockSpec((t, d), lambda i, f, grp, tok: (0, 0)),
            pl.BlockSpec((1, d, FB), lambda i, f, grp, tok: (grp[i], 0, f)),
            pl.BlockSpec((1, d, FB), lambda i, f, grp, tok: (grp[i], 0, f)),
            pl.BlockSpec((1, FB, d), lambda i, f, grp, tok: (grp[i], f, 0)),
            pl.BlockSpec((TILE, 1), lambda i, f, grp, tok: (i, 0)),
        ],
        out_specs=pl.BlockSpec((t, d), lambda i, f, grp, tok: (0, 0)),
        scratch_shapes=[
            pltpu.VMEM((TILE, d), jnp.float32),
            pltpu.VMEM((TILE, d), jnp.float32),
            pltpu.VMEM((TILE, d), jnp.float32),
        ],
    )

    out = pl.pallas_call(
        functools.partial(_gmm_kernel, nf=nf),
        grid_spec=grid_spec,
        out_shape=jax.ShapeDtypeStruct((t, d), jnp.float32),
        compiler_params=pltpu.CompilerParams(
            dimension_semantics=("arbitrary", "arbitrary"),
        ),
    )(tile_grp, sorted_tok, x, wi_0, wi_1, wo, sorted_w.reshape(padrows, 1))

    return out.reshape(b, s, d)
